# baseline (speedup 1.0000x reference)
import functools

import jax
import jax.numpy as jnp
from jax import lax
from jax.experimental import pallas as pl
from jax.experimental.pallas import tpu as pltpu
from jax.experimental.pallas import tpu_sc as plsc

NS = 16
CH = 125
WPT = 10
HD = 64


def _sc_body(NJ, N, xu_h, xi_h, src_h, dst_h, zeros_h, zeros8_h, ones_h,
             sum_h, cnt_h,
             sidx, didx, r0, r1, r2, r3, r4, obuf, acc, cacc,
             gsem, ssem, csem):
    rbufs = (r0, r1, r2, r3, r4)
    NR = 5
    LAG = 4
    SLAG = NR - LAG
    c = lax.axis_index("c")
    s = lax.axis_index("s")
    rpw = N // WPT

    pltpu.sync_copy(ones_h, obuf)

    pltpu.sync_copy(dst_h.at[c, s], didx)

    base = s * rpw
    obase = (1 - c) * N + base

    for h in range(2):
        @pl.when(s < WPT)
        def _():
            pltpu.sync_copy(zeros_h, acc.at[pl.ds(base, rpw)])
            if h == 0:
                pltpu.sync_copy(zeros8_h, cacc.at[pl.ds(base, rpw)])

        pltpu.sync_copy(src_h.at[h, c, s], sidx)

        plsc.subcore_barrier()

        def pipeline(tab):
            for p in range(LAG):
                pltpu.async_copy(tab.at[sidx.at[p]], rbufs[p], gsem)

            def step(g, carry):
                for b in range(NR):
                    j = NR * g + b
                    r = rbufs[b]
                    rn = rbufs[(b + LAG) % NR]
                    pltpu.make_async_copy(tab.at[sidx.at[j]], r,
                                          gsem).wait()
                    pltpu.async_copy(r, acc.at[didx.at[j]], ssem, add=True)
                    if h == 0:
                        pltpu.async_copy(obuf, cacc.at[didx.at[j]], csem,
                                         add=True)

                    @pl.when(j >= SLAG)
                    def _():
                        pltpu.make_async_copy(rn, acc.at[didx.at[j]],
                                              ssem).wait()
                        if h == 0:
                            pltpu.make_async_copy(obuf,
                                                  cacc.at[didx.at[j]],
                                                  csem).wait()

                    @pl.when(j + LAG < NJ)
                    def _():
                        pltpu.async_copy(tab.at[sidx.at[j + LAG]], rn,
                                         gsem)
                return carry

            lax.fori_loop(0, NJ // NR, step, 0)

            for _ in range(SLAG):
                pltpu.make_async_copy(rbufs[0], acc.at[didx.at[0]],
                                      ssem).wait()
                if h == 0:
                    pltpu.make_async_copy(obuf, cacc.at[didx.at[0]],
                                          csem).wait()

        @pl.when(c == 0)
        def _():
            pipeline(xu_h)

        @pl.when(c == 1)
        def _():
            pipeline(xi_h)

        plsc.subcore_barrier()

        @pl.when(s < WPT)
        def _():
            pltpu.sync_copy(acc.at[pl.ds(base, rpw)],
                            sum_h.at[h, pl.ds(obase, rpw)])
            if h == 0:
                pltpu.sync_copy(cacc.at[pl.ds(base, rpw)],
                                cnt_h.at[pl.ds(obase, rpw)])
        if h == 0:
            plsc.subcore_barrier()


def _make_sc_scatter(N, E):
    EPT = E // NS
    NJ = EPT // CH
    mesh = plsc.VectorSubcoreMesh(core_axis_name="c", subcore_axis_name="s",
                                  num_cores=2, num_subcores=NS)
    return pl.kernel(
        functools.partial(_sc_body, NJ, N),
        out_type=[
            jax.ShapeDtypeStruct((2, 2 * N, HD), jnp.float32),
            jax.ShapeDtypeStruct((2 * N, 8), jnp.float32),
        ],
        mesh=mesh,
        scratch_types=[
            pltpu.VMEM((NJ, CH), jnp.int32),
            pltpu.VMEM((NJ, CH), jnp.int32),
        ] + [pltpu.VMEM((CH, HD), jnp.float32)] * 5 + [
            pltpu.VMEM((CH, 8), jnp.float32),
            pltpu.VMEM_SHARED((N, HD), jnp.float32),
            pltpu.VMEM_SHARED((N, 8), jnp.float32),
            pltpu.SemaphoreType.DMA,
            pltpu.SemaphoreType.DMA,
            pltpu.SemaphoreType.DMA,
        ],
        compiler_params=pltpu.CompilerParams(use_tc_tiling_on_sc=False),
    )


def _tc_body(s0_ref, s1_ref, c_ref, x_ref,
             wl_ref, bl_ref, wr_ref, g_ref, b_ref, o_ref):
    ssum = jnp.concatenate([s0_ref[0], s1_ref[0]], axis=-1)
    cnt = c_ref[:, 0:1]
    x = x_ref[...]
    agg = jnp.where(cnt > 0, ssum / jnp.maximum(cnt, 1.0), 0.0)
    h = (jnp.dot(agg, wl_ref[...], preferred_element_type=jnp.float32)
         + bl_ref[...]
         + jnp.dot(x, wr_ref[...], preferred_element_type=jnp.float32))
    mu = jnp.mean(h, axis=-1, keepdims=True)
    var = jnp.mean((h - mu) * (h - mu), axis=-1, keepdims=True)
    y = (h - mu) / jnp.sqrt(var + 1e-5) * g_ref[...] + b_ref[...]
    o_ref[...] = jnp.maximum(y, 0.0)


def _make_tc_dense(N, toff, BLK=1000):
    NB = N // BLK

    def srow(h):
        return lambda i, _h=h, _o=toff * NB: (_h, _o + i, 0)

    crow = lambda i, _o=toff * NB: (_o + i, 0)
    row = lambda i: (i, 0)
    par = lambda i: (0, 0)
    return pl.pallas_call(
        _tc_body,
        grid=(NB,),
        in_specs=[
            pl.BlockSpec((1, BLK, HD), srow(0)),
            pl.BlockSpec((1, BLK, HD), srow(1)),
            pl.BlockSpec((BLK, 8), crow),
            pl.BlockSpec((BLK, 128), row),
            pl.BlockSpec((128, 128), par),
            pl.BlockSpec((1, 128), par),
            pl.BlockSpec((128, 128), par),
            pl.BlockSpec((1, 128), par),
            pl.BlockSpec((1, 128), par),
        ],
        out_specs=pl.BlockSpec((BLK, 128), row),
        out_shape=jax.ShapeDtypeStruct((N, 128), jnp.float32),
    )


def kernel(x_user, x_item, ei_u2i, ei_i2u,
           Wl0_u2i, bl0_u2i, Wr0_u2i, Wl0_i2u, bl0_i2u, Wr0_i2u,
           Wl1_u2i, bl1_u2i, Wr1_u2i, Wl1_i2u, bl1_i2u, Wr1_i2u,
           g0_user, b0_user, g0_item, b0_item,
           g1_user, b1_user, g1_item, b1_item):
    N = x_user.shape[0]
    E = ei_u2i.shape[1]
    EPT = E // NS
    NJ = EPT // CH

    src2 = 2 * jnp.concatenate([ei_u2i[0], ei_i2u[0]])
    src5 = jnp.stack([src2, src2 + 1]).reshape(2, 2, NS, NJ, CH)
    dst4 = jnp.concatenate([ei_u2i[1], ei_i2u[1]]).reshape(2, NS, NJ, CH)

    sc = _make_sc_scatter(N, E)
    tc_u = _make_tc_dense(N, 0)
    tc_i = _make_tc_dense(N, 1)

    layers = (
        ((Wl0_i2u, bl0_i2u, Wr0_i2u, g0_user, b0_user),
         (Wl0_u2i, bl0_u2i, Wr0_u2i, g0_item, b0_item)),
        ((Wl1_i2u, bl1_i2u, Wr1_i2u, g1_user, b1_user),
         (Wl1_u2i, bl1_u2i, Wr1_u2i, g1_item, b1_item)),
    )

    zeros = jnp.zeros((N // WPT, HD), jnp.float32)
    zeros8 = jnp.zeros((N // WPT, 8), jnp.float32)
    ones = jnp.ones((CH, 8), jnp.float32)

    x_u, x_i = x_user, x_item
    for pu, pi in layers:
        ssum, cnt = sc(x_u.reshape(2 * N, HD), x_i.reshape(2 * N, HD),
                       src5, dst4, zeros, zeros8, ones)
        x_u = tc_u(ssum, ssum, cnt, x_u,
                   pu[0], pu[1].reshape(1, 128), pu[2],
                   pu[3].reshape(1, 128), pu[4].reshape(1, 128))
        x_i = tc_i(ssum, ssum, cnt, x_i,
                   pi[0], pi[1].reshape(1, 128), pi[2],
                   pi[3].reshape(1, 128), pi[4].reshape(1, 128))
    return x_u, x_i


# speedup vs baseline: 1.0354x; 1.0354x over previous
import functools

import jax
import jax.numpy as jnp
from jax import lax
from jax.experimental import pallas as pl
from jax.experimental.pallas import tpu as pltpu
from jax.experimental.pallas import tpu_sc as plsc

NS = 16
CH = 125
WPT = 10
HD = 64


def _sc_body(NJ, N, WC, xu_h, xi_h, src_h, dst_h, zeros_h, zeros8_h, ones_h,
             sum_h, cnt_h,
             sidx, didx, r0, r1, r2, r3, r4, obuf, acc, cacc,
             gsem, ssem, csem):
    rbufs = (r0, r1, r2, r3, r4)
    NR = 5
    LAG = 3
    SLAG = NR - LAG
    c = lax.axis_index("c")
    s = lax.axis_index("s")
    rpw = N // WPT

    pltpu.sync_copy(ones_h, obuf)

    pltpu.sync_copy(dst_h.at[c, s], didx)

    base = s * rpw
    obase = (1 - c) * N + base

    for h in range(2):
        @pl.when(s < WPT)
        def _():
            pltpu.sync_copy(zeros_h, acc.at[pl.ds(base, rpw)])
            if h == 0 and WC:
                pltpu.sync_copy(zeros8_h, cacc.at[pl.ds(base, rpw)])

        pltpu.sync_copy(src_h.at[h, c, s], sidx)

        plsc.subcore_barrier()

        def pipeline(tab):
            for p in range(LAG):
                pltpu.async_copy(tab.at[sidx.at[p]], rbufs[p], gsem)

            def step(g, carry):
                for b in range(NR):
                    j = NR * g + b
                    r = rbufs[b]
                    rn = rbufs[(b + LAG) % NR]
                    pltpu.make_async_copy(tab.at[sidx.at[j]], r,
                                          gsem).wait()
                    pltpu.async_copy(r, acc.at[didx.at[j]], ssem, add=True)
                    if h == 0 and WC:
                        pltpu.async_copy(obuf, cacc.at[didx.at[j]], csem,
                                         add=True)

                    @pl.when(j >= SLAG)
                    def _():
                        pltpu.make_async_copy(rn, acc.at[didx.at[j]],
                                              ssem).wait()
                        if h == 0 and WC:
                            pltpu.make_async_copy(obuf,
                                                  cacc.at[didx.at[j]],
                                                  csem).wait()

                    @pl.when(j + LAG < NJ)
                    def _():
                        pltpu.async_copy(tab.at[sidx.at[j + LAG]], rn,
                                         gsem)
                return carry

            lax.fori_loop(0, NJ // NR, step, 0)

            for _ in range(SLAG):
                pltpu.make_async_copy(rbufs[0], acc.at[didx.at[0]],
                                      ssem).wait()
                if h == 0 and WC:
                    pltpu.make_async_copy(obuf, cacc.at[didx.at[0]],
                                          csem).wait()

        @pl.when(c == 0)
        def _():
            pipeline(xu_h)

        @pl.when(c == 1)
        def _():
            pipeline(xi_h)

        plsc.subcore_barrier()

        @pl.when(s < WPT)
        def _():
            pltpu.sync_copy(acc.at[pl.ds(base, rpw)],
                            sum_h.at[h, pl.ds(obase, rpw)])
            if h == 0 and WC:
                pltpu.sync_copy(cacc.at[pl.ds(base, rpw)],
                                cnt_h.at[pl.ds(obase, rpw)])
        if h == 0:
            plsc.subcore_barrier()


def _make_sc_scatter(N, E, wc):
    EPT = E // NS
    NJ = EPT // CH
    mesh = plsc.VectorSubcoreMesh(core_axis_name="c", subcore_axis_name="s",
                                  num_cores=2, num_subcores=NS)
    return pl.kernel(
        functools.partial(_sc_body, NJ, N, wc),
        out_type=[
            jax.ShapeDtypeStruct((2, 2 * N, HD), jnp.float32),
            jax.ShapeDtypeStruct((2 * N, 8), jnp.float32),
        ],
        mesh=mesh,
        scratch_types=[
            pltpu.VMEM((NJ, CH), jnp.int32),
            pltpu.VMEM((NJ, CH), jnp.int32),
        ] + [pltpu.VMEM((CH, HD), jnp.float32)] * 5 + [
            pltpu.VMEM((CH, 8), jnp.float32),
            pltpu.VMEM_SHARED((N, HD), jnp.float32),
            pltpu.VMEM_SHARED((N, 8), jnp.float32),
            pltpu.SemaphoreType.DMA,
            pltpu.SemaphoreType.DMA,
            pltpu.SemaphoreType.DMA,
        ],
        compiler_params=pltpu.CompilerParams(use_tc_tiling_on_sc=False),
    )


def _tc_body(s0_ref, s1_ref, c_ref, x_ref,
             wl_ref, bl_ref, wr_ref, g_ref, b_ref, o_ref):
    ssum = jnp.concatenate([s0_ref[0], s1_ref[0]], axis=-1)
    cnt = c_ref[:, 0:1]
    x = x_ref[...]
    agg = jnp.where(cnt > 0, ssum / jnp.maximum(cnt, 1.0), 0.0)
    h = (jnp.dot(agg, wl_ref[...], preferred_element_type=jnp.float32)
         + bl_ref[...]
         + jnp.dot(x, wr_ref[...], preferred_element_type=jnp.float32))
    mu = jnp.mean(h, axis=-1, keepdims=True)
    var = jnp.mean((h - mu) * (h - mu), axis=-1, keepdims=True)
    y = (h - mu) / jnp.sqrt(var + 1e-5) * g_ref[...] + b_ref[...]
    o_ref[...] = jnp.maximum(y, 0.0)


def _make_tc_dense(N, toff, BLK=1000):
    NB = N // BLK

    def srow(h):
        return lambda i, _h=h, _o=toff * NB: (_h, _o + i, 0)

    crow = lambda i, _o=toff * NB: (_o + i, 0)
    row = lambda i: (i, 0)
    par = lambda i: (0, 0)
    return pl.pallas_call(
        _tc_body,
        grid=(NB,),
        in_specs=[
            pl.BlockSpec((1, BLK, HD), srow(0)),
            pl.BlockSpec((1, BLK, HD), srow(1)),
            pl.BlockSpec((BLK, 8), crow),
            pl.BlockSpec((BLK, 128), row),
            pl.BlockSpec((128, 128), par),
            pl.BlockSpec((1, 128), par),
            pl.BlockSpec((128, 128), par),
            pl.BlockSpec((1, 128), par),
            pl.BlockSpec((1, 128), par),
        ],
        out_specs=pl.BlockSpec((BLK, 128), row),
        out_shape=jax.ShapeDtypeStruct((N, 128), jnp.float32),
    )


def kernel(x_user, x_item, ei_u2i, ei_i2u,
           Wl0_u2i, bl0_u2i, Wr0_u2i, Wl0_i2u, bl0_i2u, Wr0_i2u,
           Wl1_u2i, bl1_u2i, Wr1_u2i, Wl1_i2u, bl1_i2u, Wr1_i2u,
           g0_user, b0_user, g0_item, b0_item,
           g1_user, b1_user, g1_item, b1_item):
    N = x_user.shape[0]
    E = ei_u2i.shape[1]
    EPT = E // NS
    NJ = EPT // CH

    src2 = 2 * jnp.concatenate([ei_u2i[0], ei_i2u[0]])
    src5 = jnp.stack([src2, src2 + 1]).reshape(2, 2, NS, NJ, CH)
    dst4 = jnp.concatenate([ei_u2i[1], ei_i2u[1]]).reshape(2, NS, NJ, CH)

    sc_c = _make_sc_scatter(N, E, True)
    sc_n = _make_sc_scatter(N, E, False)
    tc_u = _make_tc_dense(N, 0)
    tc_i = _make_tc_dense(N, 1)

    layers = (
        ((Wl0_i2u, bl0_i2u, Wr0_i2u, g0_user, b0_user),
         (Wl0_u2i, bl0_u2i, Wr0_u2i, g0_item, b0_item)),
        ((Wl1_i2u, bl1_i2u, Wr1_i2u, g1_user, b1_user),
         (Wl1_u2i, bl1_u2i, Wr1_u2i, g1_item, b1_item)),
    )

    zeros = jnp.zeros((N // WPT, HD), jnp.float32)
    zeros8 = jnp.zeros((N // WPT, 8), jnp.float32)
    ones = jnp.ones((CH, 8), jnp.float32)

    x_u, x_i = x_user, x_item
    cnt = None
    for li, (pu, pi) in enumerate(layers):
        sc = sc_c if li == 0 else sc_n
        ssum, cnt_l = sc(x_u.reshape(2 * N, HD), x_i.reshape(2 * N, HD),
                         src5, dst4, zeros, zeros8, ones)
        if li == 0:
            cnt = cnt_l
        x_u = tc_u(ssum, ssum, cnt, x_u,
                   pu[0], pu[1].reshape(1, 128), pu[2],
                   pu[3].reshape(1, 128), pu[4].reshape(1, 128))
        x_i = tc_i(ssum, ssum, cnt, x_i,
                   pi[0], pi[1].reshape(1, 128), pi[2],
                   pi[3].reshape(1, 128), pi[4].reshape(1, 128))
    return x_u, x_i
